# baseline (speedup 1.0000x reference)
import functools

import jax
import jax.numpy as jnp
from jax import lax
from jax.experimental import pallas as pl
from jax.experimental.pallas import tpu as pltpu
from jax.experimental.pallas import tpu_sc as plsc

N = 10000
E = 320000
NEG = 0.2

NC = 2
NS = 16
NW = NC * NS
EPW = E // NW
CH = 80
NCHUNK = EPW // CH
ZR = 40
NZC = N // ZR
NZI = (NZC + NS - 1) // NS

D0 = 144
D1 = 80


def _leaky_exp(e):
    e = jnp.where(e > 0.0, e, NEG * e)
    return jnp.exp(e)


_GD = lax.GatherDimensionNumbers(
    offset_dims=(), collapsed_slice_dims=(0,), start_index_map=(0,))


def _splat(v, idx):
    return lax.gather(v, idx[:, None], _GD, slice_sizes=(1,),
                      mode=lax.GatherScatterMode.PROMISE_IN_BOUNDS)


def _tc_proj0(feat, W0, al0f, ar0f, seg, F0, er0p):
    ft = jnp.dot(feat[...], W0[...], preferred_element_type=jnp.float32)
    el = jnp.dot(ft * al0f[...], seg[...], preferred_element_type=jnp.float32)
    er = jnp.dot(ft * ar0f[...], seg[...], preferred_element_type=jnp.float32)
    F0[:, 0:128] = ft
    F0[:, 128:136] = el
    F0[:, 136:144] = jnp.zeros((N, 8), jnp.float32)
    er0p[:, 0:8] = er
    er0p[:, 8:16] = jnp.zeros((N, 8), jnp.float32)


def _tc_mid(acc, W1, al1c, ar1c, resW1, segt, F1, er1p, res):
    a = acc[0] + acc[1]
    num = a[:, 0:128]
    den = a[:, 128:136]
    denw = jnp.dot(den, segt[...], preferred_element_type=jnp.float32)
    h0 = num / (denw + 1e-9)
    h0 = jnp.where(h0 > 0.0, h0, jnp.exp(h0) - 1.0)
    ft1 = jnp.dot(h0, W1[...], preferred_element_type=jnp.float32)
    el1 = jnp.dot(ft1, al1c[...], preferred_element_type=jnp.float32)
    er1 = jnp.dot(ft1, ar1c[...], preferred_element_type=jnp.float32)
    F1[:, 0:64] = ft1
    F1[:, 64:65] = el1
    F1[:, 65:80] = jnp.zeros((N, 15), jnp.float32)
    er1p[:, 0:1] = er1
    er1p[:, 1:16] = jnp.zeros((N, 15), jnp.float32)
    res[...] = jnp.dot(h0, resW1[...], preferred_element_type=jnp.float32)


def _tc_final(acc, res, out):
    a = acc[0] + acc[1]
    out[...] = a[:, 0:64] / (a[:, 64:65] + 1e-9) + res[...]


def _sc_layer0(F0, erp, srcr, dstr, out, acc, ids_s, ids_d, G, R, zb, s1, s2):
    c = lax.axis_index("c")
    s = lax.axis_index("s")
    wid = c * NS + s

    def zrow(i, _):
        def zcol(j, _):
            zb[i, pl.ds(j * 16, 16)] = jnp.zeros((16,), jnp.float32)
            return 0
        return lax.fori_loop(0, D0 // 16, zcol, 0)
    lax.fori_loop(0, ZR, zrow, 0)

    def zacc(i, _):
        j = s + i * NS

        @pl.when(j < NZC)
        def _():
            pltpu.sync_copy(zb, acc.at[pl.ds(j * ZR, ZR)])
        return 0
    lax.fori_loop(0, NZI, zacc, 0)
    plsc.subcore_barrier()

    pltpu.sync_copy(srcr.at[wid], ids_s)
    pltpu.sync_copy(dstr.at[wid], ids_d)

    hvecs = [jnp.full((16,), h, jnp.int32) for h in range(8)]

    def chunk(i, _):
        cp1 = pltpu.async_copy(F0.at[ids_s.at[i]], G, s1)
        cp2 = pltpu.async_copy(erp.at[ids_d.at[i]], R, s2)
        cp1.wait()
        cp2.wait()

        @plsc.parallel_loop(0, CH, unroll=4)
        def edge(k):
            ev = G[k, pl.ds(128, 16)]
            rv = R[k, pl.ds(0, 16)]
            w = _leaky_exp(ev + rv)
            G[k, pl.ds(128, 16)] = w
            for h in range(8):
                wh = _splat(w, hvecs[h])
                G[k, pl.ds(h * 16, 16)] = G[k, pl.ds(h * 16, 16)] * wh

        pltpu.sync_copy(G, acc.at[ids_d.at[i]], add=True)
        return 0
    lax.fori_loop(0, NCHUNK, chunk, 0)
    plsc.subcore_barrier()

    def outp(i, _):
        j = s + i * NS

        @pl.when(j < NZC)
        def _():
            pltpu.sync_copy(acc.at[pl.ds(j * ZR, ZR)], zb)
            pltpu.sync_copy(zb, out.at[c, pl.ds(j * ZR, ZR)])
        return 0
    lax.fori_loop(0, NZI, outp, 0)


def _sc_layer1(F1, erp, srcr, dstr, out, acc, ids_s, ids_d, G, R, zb, s1, s2):
    c = lax.axis_index("c")
    s = lax.axis_index("s")
    wid = c * NS + s

    def zrow(i, _):
        def zcol(j, _):
            zb[i, pl.ds(j * 16, 16)] = jnp.zeros((16,), jnp.float32)
            return 0
        return lax.fori_loop(0, D1 // 16, zcol, 0)
    lax.fori_loop(0, ZR, zrow, 0)

    def zacc(i, _):
        j = s + i * NS

        @pl.when(j < NZC)
        def _():
            pltpu.sync_copy(zb, acc.at[pl.ds(j * ZR, ZR)])
        return 0
    lax.fori_loop(0, NZI, zacc, 0)
    plsc.subcore_barrier()

    pltpu.sync_copy(srcr.at[wid], ids_s)
    pltpu.sync_copy(dstr.at[wid], ids_d)

    lanes = lax.iota(jnp.int32, 16)
    c64 = jnp.full((16,), 64, jnp.int32)
    c0 = jnp.full((16,), 0, jnp.int32)

    def chunk(i, _):
        cp1 = pltpu.async_copy(F1.at[ids_s.at[i]], G, s1)
        cp2 = pltpu.async_copy(erp.at[ids_d.at[i]], R, s2)
        cp1.wait()
        cp2.wait()

        @plsc.parallel_loop(0, CH // 16)
        def att(j):
            rows = lanes + j * 16
            elv = plsc.load_gather(G, [rows, c64])
            erv = plsc.load_gather(R, [rows, c0])
            w = _leaky_exp(elv + erv)
            plsc.store_scatter(G, [rows, c64], w)

        @plsc.parallel_loop(0, CH, unroll=4)
        def edge(k):
            wv = G[k, pl.ds(64, 16)]
            wh = _splat(wv, c0)
            for q in range(4):
                G[k, pl.ds(q * 16, 16)] = G[k, pl.ds(q * 16, 16)] * wh

        pltpu.sync_copy(G, acc.at[ids_d.at[i]], add=True)
        return 0
    lax.fori_loop(0, NCHUNK, chunk, 0)
    plsc.subcore_barrier()

    def outp(i, _):
        j = s + i * NS

        @pl.when(j < NZC)
        def _():
            pltpu.sync_copy(acc.at[pl.ds(j * ZR, ZR)], zb)
            pltpu.sync_copy(zb, out.at[c, pl.ds(j * ZR, ZR)])
        return 0
    lax.fori_loop(0, NZI, outp, 0)


_MESH = plsc.VectorSubcoreMesh(core_axis_name="c", subcore_axis_name="s")
_SC_PARAMS = pltpu.CompilerParams(
    use_tc_tiling_on_sc=False, needs_layout_passes=False)

_sc0_call = pl.kernel(
    _sc_layer0,
    out_type=jax.ShapeDtypeStruct((NC, N, D0), jnp.float32),
    mesh=_MESH,
    scratch_types=[
        pltpu.VMEM_SHARED((N, D0), jnp.float32),
        pltpu.VMEM((NCHUNK, CH), jnp.int32),
        pltpu.VMEM((NCHUNK, CH), jnp.int32),
        pltpu.VMEM((CH, D0), jnp.float32),
        pltpu.VMEM((CH, 16), jnp.float32),
        pltpu.VMEM((ZR, D0), jnp.float32),
        pltpu.SemaphoreType.DMA,
        pltpu.SemaphoreType.DMA,
    ],
    compiler_params=_SC_PARAMS,
)

_sc1_call = pl.kernel(
    _sc_layer1,
    out_type=jax.ShapeDtypeStruct((NC, N, D1), jnp.float32),
    mesh=_MESH,
    scratch_types=[
        pltpu.VMEM_SHARED((N, D1), jnp.float32),
        pltpu.VMEM((NCHUNK, CH), jnp.int32),
        pltpu.VMEM((NCHUNK, CH), jnp.int32),
        pltpu.VMEM((CH, D1), jnp.float32),
        pltpu.VMEM((CH, 16), jnp.float32),
        pltpu.VMEM((ZR, D1), jnp.float32),
        pltpu.SemaphoreType.DMA,
        pltpu.SemaphoreType.DMA,
    ],
    compiler_params=_SC_PARAMS,
)


def kernel(feat, edge_index, W0, al0, ar0, W1, al1, ar1, resW1):
    src = edge_index[0].astype(jnp.int32).reshape(NW, NCHUNK, CH)
    dst = edge_index[1].astype(jnp.int32).reshape(NW, NCHUNK, CH)

    seg = jnp.repeat(jnp.eye(8, dtype=jnp.float32), 16, axis=0)
    segt = seg.T
    al0f = al0.reshape(1, 128)
    ar0f = ar0.reshape(1, 128)
    al1c = al1.reshape(64, 1)
    ar1c = ar1.reshape(64, 1)

    F0, er0p = pl.pallas_call(
        _tc_proj0,
        out_shape=[
            jax.ShapeDtypeStruct((N, D0), jnp.float32),
            jax.ShapeDtypeStruct((N, 16), jnp.float32),
        ],
    )(feat, W0, al0f, ar0f, seg)

    acc0 = _sc0_call(F0, er0p, src, dst)

    F1, er1p, res = pl.pallas_call(
        _tc_mid,
        out_shape=[
            jax.ShapeDtypeStruct((N, D1), jnp.float32),
            jax.ShapeDtypeStruct((N, 16), jnp.float32),
            jax.ShapeDtypeStruct((N, 64), jnp.float32),
        ],
    )(acc0, W1, al1c, ar1c, resW1, segt)

    acc1 = _sc1_call(F1, er1p, src, dst)

    out = pl.pallas_call(
        _tc_final,
        out_shape=jax.ShapeDtypeStruct((N, 64), jnp.float32),
    )(acc1, res)
    return out


# speedup vs baseline: 1.4244x; 1.4244x over previous
import jax
import jax.numpy as jnp
from jax import lax
from jax.experimental import pallas as pl
from jax.experimental.pallas import tpu as pltpu
from jax.experimental.pallas import tpu_sc as plsc

N = 10000
E = 320000
NEG = 0.2

NC = 2
NS = 16
NW = NC * NS
EPW = E // NW
CH = 80
NCHUNK = EPW // CH
NPAIR = (NCHUNK + 1) // 2
NZC = N // CH
NZI = (NZC + NS - 1) // NS

D0 = 136
D1 = 80


def _leaky_exp(e):
    e = jnp.where(e > 0.0, e, NEG * e)
    return jnp.exp(e)


_GD = lax.GatherDimensionNumbers(
    offset_dims=(), collapsed_slice_dims=(0,), start_index_map=(0,))


def _splat(v, idx):
    return lax.gather(v, idx[:, None], _GD, slice_sizes=(1,),
                      mode=lax.GatherScatterMode.PROMISE_IN_BOUNDS)


def _tc_proj0(feat, W0, al0f, ar0f, seg, F0, er0p):
    ft = jnp.dot(feat[...], W0[...], preferred_element_type=jnp.float32)
    el = jnp.dot(ft * al0f[...], seg[...], preferred_element_type=jnp.float32)
    er = jnp.dot(ft * ar0f[...], seg[...], preferred_element_type=jnp.float32)
    F0[:, 0:128] = ft
    F0[:, 128:136] = el
    er0p[:, 0:8] = jnp.zeros((N, 8), jnp.float32)
    er0p[:, 8:16] = er


def _tc_mid(acc, W1, al1c, ar1c, resW1, segt, F1, er1p, res):
    a = acc[0] + acc[1]
    num = a[:, 0:128]
    den = a[:, 128:136]
    denw = jnp.dot(den, segt[...], preferred_element_type=jnp.float32)
    h0 = num / (denw + 1e-9)
    h0 = jnp.where(h0 > 0.0, h0, jnp.exp(h0) - 1.0)
    ft1 = jnp.dot(h0, W1[...], preferred_element_type=jnp.float32)
    el1 = jnp.dot(ft1, al1c[...], preferred_element_type=jnp.float32)
    er1 = jnp.dot(ft1, ar1c[...], preferred_element_type=jnp.float32)
    F1[:, 0:64] = ft1
    F1[:, 64:65] = el1
    F1[:, 65:80] = jnp.zeros((N, 15), jnp.float32)
    er1p[:, 0:1] = er1
    er1p[:, 1:16] = jnp.zeros((N, 15), jnp.float32)
    res[...] = jnp.dot(h0, resW1[...], preferred_element_type=jnp.float32)


def _tc_final(acc, res, out):
    a = acc[0] + acc[1]
    out[...] = a[:, 0:64] / (a[:, 64:65] + 1e-9) + res[...]


def _zero_buf(G, d):
    @plsc.parallel_loop(0, CH)
    def zrow(i):
        for j in range(d // 16):
            G[i, pl.ds(j * 16, 16)] = jnp.zeros((16,), jnp.float32)
        if d % 16:
            G[i, pl.ds(d - 16, 16)] = jnp.zeros((16,), jnp.float32)


def _zero_acc(G, acc, s):
    def zacc(i, _):
        j = s + i * NS

        @pl.when(j < NZC)
        def _():
            pltpu.sync_copy(G, acc.at[pl.ds(j * CH, CH)])
        return 0
    lax.fori_loop(0, NZI, zacc, 0)


def _acc_out(acc, out, c, s):
    def outp(i, _):
        j = s + i * NS

        @pl.when(j < NZC)
        def _():
            pltpu.sync_copy(acc.at[pl.ds(j * CH, CH)],
                            out.at[c, pl.ds(j * CH, CH)])
        return 0
    lax.fori_loop(0, NZI, outp, 0)


def _sc_layer0(F0, erp, srcr, dstr, out, acc, ids_s, ids_d,
               Ga, Gb, Ra, Rb, s1a, s2a, s1b, s2b):
    c = lax.axis_index("c")
    s = lax.axis_index("s")
    wid = c * NS + s

    _zero_buf(Ga, D0)
    _zero_acc(Ga, acc, s)
    plsc.subcore_barrier()

    pltpu.sync_copy(srcr.at[wid], ids_s)
    pltpu.sync_copy(dstr.at[wid], ids_d)

    hvecs = [jnp.full((16,), 8 + h, jnp.int32) for h in range(8)]
    lanes = lax.iota(jnp.int32, 16)
    cw = lanes + 120
    wmask = lanes >= 8

    def start_g(i, G, R, sg, sr):
        pltpu.make_async_copy(F0.at[ids_s.at[i]], G, sg).start()
        pltpu.make_async_copy(erp.at[ids_d.at[i]], R, sr).start()

    def wait_g(i, G, R, sg, sr):
        pltpu.make_async_copy(F0.at[ids_s.at[i]], G, sg).wait()
        pltpu.make_async_copy(erp.at[ids_d.at[i]], R, sr).wait()

    def compute(G, R):
        @plsc.parallel_loop(0, CH, unroll=2)
        def edge(k):
            ev = G[k, pl.ds(120, 16)]
            rv = R[k, pl.ds(0, 16)]
            w = _leaky_exp(ev + rv)
            kvec = jnp.full((16,), k, jnp.int32)
            plsc.store_scatter(G, [kvec, cw], w, mask=wmask)
            for h in range(8):
                wh = _splat(w, hvecs[h])
                G[k, pl.ds(h * 16, 16)] = G[k, pl.ds(h * 16, 16)] * wh

    start_g(0, Ga, Ra, s1a, s2a)

    def pair(p, _):
        i = 2 * p
        wait_g(i, Ga, Ra, s1a, s2a)

        @pl.when(i + 1 < NCHUNK)
        def _():
            start_g(i + 1, Gb, Rb, s1b, s2b)
        compute(Ga, Ra)
        pltpu.sync_copy(Ga, acc.at[ids_d.at[i]], add=True)

        @pl.when(i + 1 < NCHUNK)
        def _():
            wait_g(i + 1, Gb, Rb, s1b, s2b)

            @pl.when(i + 2 < NCHUNK)
            def _():
                start_g(i + 2, Ga, Ra, s1a, s2a)
            compute(Gb, Rb)
            pltpu.sync_copy(Gb, acc.at[ids_d.at[i + 1]], add=True)
        return 0
    lax.fori_loop(0, NPAIR, pair, 0)
    plsc.subcore_barrier()

    _acc_out(acc, out, c, s)


def _sc_layer1(F1, erp, srcr, dstr, out, acc, ids_s, ids_d,
               Ga, Gb, Ra, Rb, s1a, s2a, s1b, s2b):
    c = lax.axis_index("c")
    s = lax.axis_index("s")
    wid = c * NS + s

    _zero_buf(Ga, D1)
    _zero_acc(Ga, acc, s)
    plsc.subcore_barrier()

    pltpu.sync_copy(srcr.at[wid], ids_s)
    pltpu.sync_copy(dstr.at[wid], ids_d)

    lanes = lax.iota(jnp.int32, 16)
    c64 = jnp.full((16,), 64, jnp.int32)
    c0 = jnp.full((16,), 0, jnp.int32)

    def start_g(i, G, R, sg, sr):
        pltpu.make_async_copy(F1.at[ids_s.at[i]], G, sg).start()
        pltpu.make_async_copy(erp.at[ids_d.at[i]], R, sr).start()

    def wait_g(i, G, R, sg, sr):
        pltpu.make_async_copy(F1.at[ids_s.at[i]], G, sg).wait()
        pltpu.make_async_copy(erp.at[ids_d.at[i]], R, sr).wait()

    def compute(G, R):
        @plsc.parallel_loop(0, CH // 16)
        def att(j):
            rows = lanes + j * 16
            elv = plsc.load_gather(G, [rows, c64])
            erv = plsc.load_gather(R, [rows, c0])
            w = _leaky_exp(elv + erv)
            plsc.store_scatter(G, [rows, c64], w)

        @plsc.parallel_loop(0, CH, unroll=2)
        def edge(k):
            wv = G[k, pl.ds(64, 16)]
            wh = _splat(wv, c0)
            for q in range(4):
                G[k, pl.ds(q * 16, 16)] = G[k, pl.ds(q * 16, 16)] * wh

    start_g(0, Ga, Ra, s1a, s2a)

    def pair(p, _):
        i = 2 * p
        wait_g(i, Ga, Ra, s1a, s2a)

        @pl.when(i + 1 < NCHUNK)
        def _():
            start_g(i + 1, Gb, Rb, s1b, s2b)
        compute(Ga, Ra)
        pltpu.sync_copy(Ga, acc.at[ids_d.at[i]], add=True)

        @pl.when(i + 1 < NCHUNK)
        def _():
            wait_g(i + 1, Gb, Rb, s1b, s2b)

            @pl.when(i + 2 < NCHUNK)
            def _():
                start_g(i + 2, Ga, Ra, s1a, s2a)
            compute(Gb, Rb)
            pltpu.sync_copy(Gb, acc.at[ids_d.at[i + 1]], add=True)
        return 0
    lax.fori_loop(0, NPAIR, pair, 0)
    plsc.subcore_barrier()

    _acc_out(acc, out, c, s)


_MESH = plsc.VectorSubcoreMesh(core_axis_name="c", subcore_axis_name="s")
_SC_PARAMS = pltpu.CompilerParams(
    use_tc_tiling_on_sc=False, needs_layout_passes=False)

_sc0_call = pl.kernel(
    _sc_layer0,
    out_type=jax.ShapeDtypeStruct((NC, N, D0), jnp.float32),
    mesh=_MESH,
    scratch_types=[
        pltpu.VMEM_SHARED((N, D0), jnp.float32),
        pltpu.VMEM((NCHUNK, CH), jnp.int32),
        pltpu.VMEM((NCHUNK, CH), jnp.int32),
        pltpu.VMEM((CH, D0), jnp.float32),
        pltpu.VMEM((CH, D0), jnp.float32),
        pltpu.VMEM((CH, 16), jnp.float32),
        pltpu.VMEM((CH, 16), jnp.float32),
        pltpu.SemaphoreType.DMA,
        pltpu.SemaphoreType.DMA,
        pltpu.SemaphoreType.DMA,
        pltpu.SemaphoreType.DMA,
    ],
    compiler_params=_SC_PARAMS,
)

_sc1_call = pl.kernel(
    _sc_layer1,
    out_type=jax.ShapeDtypeStruct((NC, N, D1), jnp.float32),
    mesh=_MESH,
    scratch_types=[
        pltpu.VMEM_SHARED((N, D1), jnp.float32),
        pltpu.VMEM((NCHUNK, CH), jnp.int32),
        pltpu.VMEM((NCHUNK, CH), jnp.int32),
        pltpu.VMEM((CH, D1), jnp.float32),
        pltpu.VMEM((CH, D1), jnp.float32),
        pltpu.VMEM((CH, 16), jnp.float32),
        pltpu.VMEM((CH, 16), jnp.float32),
        pltpu.SemaphoreType.DMA,
        pltpu.SemaphoreType.DMA,
        pltpu.SemaphoreType.DMA,
        pltpu.SemaphoreType.DMA,
    ],
    compiler_params=_SC_PARAMS,
)


def kernel(feat, edge_index, W0, al0, ar0, W1, al1, ar1, resW1):
    src = edge_index[0].astype(jnp.int32).reshape(NW, NCHUNK, CH)
    dst = edge_index[1].astype(jnp.int32).reshape(NW, NCHUNK, CH)

    seg = jnp.repeat(jnp.eye(8, dtype=jnp.float32), 16, axis=0)
    segt = seg.T
    al0f = al0.reshape(1, 128)
    ar0f = ar0.reshape(1, 128)
    al1c = al1.reshape(64, 1)
    ar1c = ar1.reshape(64, 1)

    F0, er0p = pl.pallas_call(
        _tc_proj0,
        out_shape=[
            jax.ShapeDtypeStruct((N, D0), jnp.float32),
            jax.ShapeDtypeStruct((N, 16), jnp.float32),
        ],
    )(feat, W0, al0f, ar0f, seg)

    acc0 = _sc0_call(F0, er0p, src, dst)

    F1, er1p, res = pl.pallas_call(
        _tc_mid,
        out_shape=[
            jax.ShapeDtypeStruct((N, D1), jnp.float32),
            jax.ShapeDtypeStruct((N, 16), jnp.float32),
            jax.ShapeDtypeStruct((N, 64), jnp.float32),
        ],
    )(acc0, W1, al1c, ar1c, resW1, segt)

    acc1 = _sc1_call(F1, er1p, src, dst)

    out = pl.pallas_call(
        _tc_final,
        out_shape=jax.ShapeDtypeStruct((N, 64), jnp.float32),
    )(acc1, res)
    return out
